# baseline (speedup 1.0000x reference)
from functools import partial

import jax
import jax.numpy as jnp
import numpy as np
from jax.experimental import pallas as pl
from jax.experimental.shard_map import shard_map
from jax.sharding import Mesh, PartitionSpec as P

_B, _T = 16, 16000
_N_IN, _N_OUT = 256, 512
_TT = 3200
_NT = _T // _TT


def _onehot_kernel(x_ref, tab_ref, out_ref):
    idx = x_ref[0, 0, 0, :]
    tab = tab_ref[...]
    ss = tab / (1.0 + jnp.abs(tab))
    iota = jax.lax.broadcasted_iota(jnp.int32, (_N_IN, _TT), 0)
    oh = (iota == idx[None, :]).astype(jnp.bfloat16)
    out = jax.lax.dot_general(
        ss.astype(jnp.bfloat16), oh,
        (((0,), (0,)), ((), ())),
        preferred_element_type=jnp.float32,
    )
    out_ref[0, :, :] = out


def _lookup(x, table):
    b = x.shape[0]
    x4 = x.astype(jnp.int32).reshape(b, _NT, 1, _TT)
    return pl.pallas_call(
        _onehot_kernel,
        grid=(b, _NT),
        in_specs=[
            pl.BlockSpec((1, 1, 1, _TT), lambda i, t: (i, t, 0, 0)),
            pl.BlockSpec((_N_IN, _N_OUT), lambda i, t: (0, 0)),
        ],
        out_specs=pl.BlockSpec((1, _N_OUT, _TT), lambda i, t: (i, 0, t)),
        out_shape=jax.ShapeDtypeStruct((b, _N_OUT, _T), jnp.float32),
    )(x4, table)


def kernel(x, table):
    nd = len(jax.devices())
    while _B % nd:
        nd -= 1
    if nd == 1:
        return _lookup(x, table)
    mesh = Mesh(np.array(jax.devices()[:nd]), ("d",))
    f = shard_map(
        _lookup, mesh=mesh,
        in_specs=(P("d", None), P(None, None)),
        out_specs=P("d", None, None),
        check_rep=False,
    )
    return f(x, table)


# speedup vs baseline: 1.1851x; 1.1851x over previous
import jax
import jax.numpy as jnp
from jax.experimental import pallas as pl

_B, _T = 16, 16000
_N_IN, _N_OUT = 256, 512
_TT = 640
_NT = _T // _TT


def _onehot_kernel(x_ref, tab_ref, out_ref):
    idx = x_ref[0, 0, 0, :]
    tab = tab_ref[...]
    ss = tab / (1.0 + jnp.abs(tab))
    iota = jax.lax.broadcasted_iota(jnp.int32, (_N_IN, _TT), 0)
    oh = (iota == idx[None, :]).astype(jnp.bfloat16)
    out = jax.lax.dot_general(
        ss.astype(jnp.bfloat16), oh,
        (((0,), (0,)), ((), ())),
        preferred_element_type=jnp.float32,
    )
    out_ref[0, :, :] = out


def _lookup(x, table):
    b = x.shape[0]
    x4 = x.astype(jnp.int32).reshape(b, _NT, 1, _TT)
    return pl.pallas_call(
        _onehot_kernel,
        grid=(b, _NT),
        in_specs=[
            pl.BlockSpec((1, 1, 1, _TT), lambda i, t: (i, t, 0, 0)),
            pl.BlockSpec((_N_IN, _N_OUT), lambda i, t: (0, 0)),
        ],
        out_specs=pl.BlockSpec((1, _N_OUT, _TT), lambda i, t: (i, 0, t)),
        out_shape=jax.ShapeDtypeStruct((b, _N_OUT, _T), jnp.float32),
    )(x4, table)


def kernel(x, table):
    return _lookup(x, table)


# speedup vs baseline: 2.4451x; 2.0632x over previous
import jax
import jax.numpy as jnp
from jax.experimental import pallas as pl

_B, _T = 16, 16000
_N_IN, _N_OUT = 256, 512
_TT = 3200
_NT = _T // _TT
_BT = 2


def _onehot_kernel(x_ref, tab_ref, out_ref):
    tab = tab_ref[...]
    ss = (tab / (1.0 + jnp.abs(tab))).astype(jnp.bfloat16)
    iota = jax.lax.broadcasted_iota(jnp.int32, (_N_IN, _TT), 0)
    for j in range(_BT):
        idx = x_ref[j, 0, 0, :]
        oh = (iota == idx[None, :]).astype(jnp.bfloat16)
        out_ref[j, :, :] = jax.lax.dot_general(
            ss, oh,
            (((0,), (0,)), ((), ())),
            preferred_element_type=jnp.float32,
        )


def _lookup(x, table):
    b = x.shape[0]
    x4 = x.astype(jnp.int32).reshape(b, _NT, 1, _TT)
    return pl.pallas_call(
        _onehot_kernel,
        grid=(b // _BT, _NT),
        in_specs=[
            pl.BlockSpec((_BT, 1, 1, _TT), lambda i, t: (i, t, 0, 0)),
            pl.BlockSpec((_N_IN, _N_OUT), lambda i, t: (0, 0)),
        ],
        out_specs=pl.BlockSpec((_BT, _N_OUT, _TT), lambda i, t: (i, 0, t)),
        out_shape=jax.ShapeDtypeStruct((b, _N_OUT, _T), jnp.float32),
    )(x4, table)


def kernel(x, table):
    return _lookup(x, table)
